# baseline (speedup 1.0000x reference)
import jax
import jax.numpy as jnp
from jax.experimental import pallas as pl
from jax.experimental.pallas import tpu as pltpu

HIDDEN = 128
RATIO = 0.5
NGRAPH = 240
NEG_SLOPE = 0.2


def _add_self_loops(ei, N):
    loops = jnp.arange(N, dtype=ei.dtype)
    return jnp.concatenate([ei[0], loops]), jnp.concatenate([ei[1], loops])


def _gat_conv(x, ei, p, heads, outc):
    N = x.shape[0]
    src, dst = _add_self_loops(ei, N)
    xl = (x @ p['W']).reshape(N, heads, outc)
    a_s = (xl * p['a_src'][None]).sum(-1)
    a_d = (xl * p['a_dst'][None]).sum(-1)
    alpha = a_s[src] + a_d[dst]
    alpha = jnp.where(alpha >= 0, alpha, NEG_SLOPE * alpha)
    m = jax.ops.segment_max(alpha, dst, num_segments=N)
    e = jnp.exp(alpha - m[dst])
    s = jax.ops.segment_sum(e, dst, num_segments=N)
    att = e / s[dst]
    out = jax.ops.segment_sum(xl[src] * att[:, :, None], dst, num_segments=N)
    return out.reshape(N, heads * outc) + p['b']


def _gatnet(x, ei, p1, p2):
    h = _gat_conv(x, ei, p1, 8, 20)
    h = jax.nn.relu(h)
    return _gat_conv(h, ei, p2, 1, HIDDEN)


def _gcn_conv(x, ei, p):
    N = x.shape[0]
    row, col = _add_self_loops(ei, N)
    deg = jax.ops.segment_sum(jnp.ones(row.shape[0], jnp.float32), col, num_segments=N)
    dis = jnp.where(deg > 0, 1.0 / jnp.sqrt(deg), 0.0)
    norm = dis[row] * dis[col]
    h = x @ p['W']
    return jax.ops.segment_sum(h[row] * norm[:, None], col, num_segments=N) + p['b']


def _graph_conv_score(x, ei, p):
    N = x.shape[0]
    agg = jax.ops.segment_sum(x[ei[0]], ei[1], num_segments=N)
    return (agg @ p['Wrel'] + p['brel'] + x @ p['Wroot']).reshape(-1)


def _sag_pool(x, ei, batch, p):
    score = _graph_conv_score(x, ei, p)
    N = x.shape[0]
    order = jnp.lexsort((-score, batch))
    counts = jax.ops.segment_sum(jnp.ones(N, jnp.int32), batch, num_segments=NGRAPH)
    starts = jnp.cumsum(counts) - counts
    batch_sorted = batch[order]
    rank = jnp.arange(N, dtype=jnp.int32) - starts[batch_sorted]
    k = jnp.ceil(RATIO * counts.astype(jnp.float32)).astype(jnp.int32)
    sel_sorted = rank < k[batch_sorted]
    selected = jnp.zeros((N,), bool).at[order].set(sel_sorted)
    v = x * jnp.tanh(score)[:, None]
    xp = jnp.where(selected[:, None], v, -jnp.inf)
    return xp, batch


def _mha_kernel(inp1_ref, inp2_ref, wq_ref, bq_ref, wk_ref, bk_ref,
                wv_ref, bv_ref, out_ref):
    q = inp1_ref[...] @ wq_ref[...] + bq_ref[...]
    k = inp2_ref[...] @ wk_ref[...] + bk_ref[...]
    v = inp2_ref[...] @ wv_ref[...] + bv_ref[...]
    hd = HIDDEN // 8
    scale = 1.0 / jnp.sqrt(jnp.float32(hd))
    for h in range(8):
        sl = slice(h * hd, (h + 1) * hd)
        qh = q[:, sl]
        kh = k[:, sl]
        vh = v[:, sl]
        scores = (qh @ kh.T) * scale
        mx = jnp.max(scores, axis=-1, keepdims=True)
        e = jnp.exp(scores - mx)
        w = e / jnp.sum(e, axis=-1, keepdims=True)
        out_ref[:, sl] = w.T @ vh


def _mha(inp1, inp2, p):
    out = pl.pallas_call(
        _mha_kernel,
        out_shape=jax.ShapeDtypeStruct((NGRAPH, HIDDEN), jnp.float32),
    )(inp1, inp2, p['Wq'], p['bq'].reshape(1, HIDDEN), p['Wk'],
      p['bk'].reshape(1, HIDDEN), p['Wv'], p['bv'].reshape(1, HIDDEN))
    return out.reshape(1, NGRAPH, HIDDEN)


def kernel(x_tree, edge_index_tree, x_graph, edge_index_graph, batch2, batch1,
           params_ast, params_dfg, params_attn):
    x = jax.nn.one_hot(x_tree, 128, dtype=jnp.float32)
    x = jax.nn.relu(_gatnet(x, edge_index_tree, params_ast['g1'], params_ast['g2']))
    x = jax.nn.relu(_gatnet(x, edge_index_tree, params_ast['g3'], params_ast['g4']))
    x, b2 = _sag_pool(x, edge_index_tree, batch2, params_ast['pool'])
    x_tf = jax.ops.segment_max(x, b2, num_segments=NGRAPH)
    y = jax.nn.one_hot(x_graph, 79, dtype=jnp.float32)
    y = jax.nn.relu(_gcn_conv(y, edge_index_graph, params_dfg['c1']))
    y = jax.nn.relu(_gcn_conv(y, edge_index_graph, params_dfg['c2']))
    y, b1 = _sag_pool(y, edge_index_graph, batch1, params_dfg['pool'])
    out_wx = jax.ops.segment_max(y, b1, num_segments=NGRAPH)
    out = _mha(out_wx, x_tf, params_attn)
    return out, out_wx
